# baseline (speedup 1.0000x reference)
import functools

import jax
import jax.numpy as jnp
from jax import lax
from jax.experimental import pallas as pl
from jax.experimental.pallas import tpu as pltpu
from jax.experimental.pallas import tpu_sc as plsc

B = 128
N = 512
L = 16
NC = 2
NS = 16
NW = NC * NS
RPW = B // NW
NCHUNK = N // L

_LN2 = 0.6931471805599453


def _vlog(k_f32):
    i = plsc.bitcast(k_f32, jnp.int32)
    e = (i >> 23) - 127
    m = plsc.bitcast((i & 0x007FFFFF) | 0x3F800000, jnp.float32)
    t = (m - 1.0) / (m + 1.0)
    t2 = t * t
    p = 1.0 / 11.0
    for c in (1.0 / 9.0, 1.0 / 7.0, 1.0 / 5.0, 1.0 / 3.0, 1.0):
        p = p * t2 + c
    return e.astype(jnp.float32) * _LN2 + 2.0 * t * p


def _vsqrt(v):
    i = plsc.bitcast(v, jnp.int32)
    y = plsc.bitcast(0x5F3759DF - (i >> 1), jnp.float32)
    for _ in range(2):
        y = y * (1.5 - 0.5 * v * y * y)
    return v * y


def _tour_body(xs_hbm, ys_hbm, out_hbm, xv, yv, ov, *sems):
    wid = lax.axis_index("c") * NS + lax.axis_index("s")
    cpx = [
        pltpu.async_copy(
            xs_hbm.at[wid, pl.ds(r * N, N)], xv.at[pl.ds(r * N, N)], sems[r]
        )
        for r in range(RPW)
    ]
    cpy = [
        pltpu.async_copy(
            ys_hbm.at[wid, pl.ds(r * N, N)], yv.at[pl.ds(r * N, N)], sems[RPW + r]
        )
        for r in range(RPW)
    ]

    lane = lax.iota(jnp.int32, L)
    zero = jnp.zeros((L,), jnp.float32)
    xv[pl.ds(RPW * N, L)] = zero
    yv[pl.ds(RPW * N, L)] = zero

    ll_acc = zero
    lane_f = lane.astype(jnp.float32)
    for c in range(N // (8 * L)):
        s = (c * L + lane_f) * 8.0 + 1.0
        p = s
        for j in range(1, 8):
            p = p * (s + j)
        ll_acc = ll_acc + _vlog(p)
    ll_s = -jnp.sum(ll_acc)

    def edge_chunk(off):
        dx = xv[pl.ds(off, L)] - xv[pl.ds(off + 1, L)]
        dy = yv[pl.ds(off, L)] - yv[pl.ds(off + 1, L)]
        return _vsqrt(dx * dx + dy * dy)

    res = jnp.where((lane >= RPW) & (lane < 2 * RPW), ll_s, zero)
    for r in range(RPW):
        base = r * N
        cpx[r].wait()
        cpy[r].wait()

        def quad(c, acc, base=base):
            off = base + c * (4 * L)
            return (acc + edge_chunk(off) + edge_chunk(off + L)
                    + edge_chunk(off + 2 * L) + edge_chunk(off + 3 * L))

        acc = lax.fori_loop(0, NCHUNK // 4 - 1, quad, zero)
        off = base + N - 4 * L
        acc = acc + edge_chunk(off) + edge_chunk(off + L) + edge_chunk(off + 2 * L)
        off = off + 3 * L
        ax = xv[pl.ds(off, L)]
        ay = yv[pl.ds(off, L)]
        firstx = xv[pl.ds(base, L)][0]
        firsty = yv[pl.ds(base, L)][0]
        bx = jnp.where(lane == L - 1, firstx, xv[pl.ds(off + 1, L)])
        by = jnp.where(lane == L - 1, firsty, yv[pl.ds(off + 1, L)])
        dx = ax - bx
        dy = ay - by
        acc = acc + _vsqrt(dx * dx + dy * dy)
        res = jnp.where(lane == r, jnp.sum(acc), res)

    ov[...] = res
    pltpu.sync_copy(ov, out_hbm.at[wid])


@functools.partial(
    pl.kernel,
    out_type=jax.ShapeDtypeStruct((NW, L), jnp.float32),
    mesh=plsc.VectorSubcoreMesh(
        core_axis_name="c", subcore_axis_name="s", num_cores=NC, num_subcores=NS
    ),
    scratch_types=(
        pltpu.VMEM((RPW * N + L,), jnp.float32),
        pltpu.VMEM((RPW * N + L,), jnp.float32),
        pltpu.VMEM((L,), jnp.float32),
    ) + (pltpu.SemaphoreType.DMA,) * (2 * RPW),
    compiler_params=pltpu.CompilerParams(
        needs_layout_passes=False,
        skip_device_barrier=True,
        disable_bounds_checks=True,
        disable_semaphore_checks=True,
    ),
)
def _tour_kernel(xs_hbm, ys_hbm, out_hbm, xv, yv, ov, *sems):
    _tour_body(xs_hbm, ys_hbm, out_hbm, xv, yv, ov, *sems)


def kernel(input):
    xs = input[:, :, 0].reshape(NW, RPW * N)
    ys = input[:, :, 1].reshape(NW, RPW * N)
    out = _tour_kernel(xs, ys)
    return out[:, :RPW].reshape(B), out[:, RPW:2 * RPW].reshape(B)


# speedup vs baseline: 1.0245x; 1.0245x over previous
import functools

import jax
import jax.numpy as jnp
from jax import lax
from jax.experimental import pallas as pl
from jax.experimental.pallas import tpu as pltpu
from jax.experimental.pallas import tpu_sc as plsc

B = 128
N = 512
L = 16
NC = 2
NS = 16
NW = NC * NS
RPW = B // NW
WPR = 2 * N
NCHUNK = WPR // L

_LN2 = 0.6931471805599453


def _vlog(k_f32):
    i = plsc.bitcast(k_f32, jnp.int32)
    e = (i >> 23) - 127
    m = plsc.bitcast((i & 0x007FFFFF) | 0x3F800000, jnp.float32)
    t = (m - 1.0) / (m + 1.0)
    t2 = t * t
    p = 1.0 / 11.0
    for c in (1.0 / 9.0, 1.0 / 7.0, 1.0 / 5.0, 1.0 / 3.0, 1.0):
        p = p * t2 + c
    return e.astype(jnp.float32) * _LN2 + 2.0 * t * p


def _vsqrt(v):
    i = plsc.bitcast(v, jnp.int32)
    y = plsc.bitcast(0x5F3759DF - (i >> 1), jnp.float32)
    for _ in range(2):
        y = y * (1.5 - 0.5 * v * y * y)
    return v * y


def _tour_body(in_hbm, out_hbm, vb, ov, *sems):
    wid = lax.axis_index("c") * NS + lax.axis_index("s")
    cps = [
        pltpu.async_copy(
            in_hbm.at[wid * RPW + r], vb.at[pl.ds(r * WPR, WPR)], sems[r]
        )
        for r in range(RPW)
    ]

    lane = lax.iota(jnp.int32, L)
    zero = jnp.zeros((L,), jnp.float32)
    even = (lane & 1) == 0
    vb[pl.ds(RPW * WPR, L)] = zero

    ll_acc = zero
    lane_f = lane.astype(jnp.float32)
    for c in range(N // (8 * L)):
        s = (c * L + lane_f) * 8.0 + 1.0
        p = s
        for j in range(1, 8):
            p = p * (s + j)
        ll_acc = ll_acc + _vlog(p)
    ll_s = -jnp.sum(ll_acc)

    def edge_group(off):
        d0 = vb[pl.ds(off, L)] - vb[pl.ds(off + 2, L)]
        d1 = vb[pl.ds(off + 1, L)] - vb[pl.ds(off + 3, L)]
        return jnp.where(even, _vsqrt(d0 * d0 + d1 * d1), 0.0)

    res = jnp.where((lane >= RPW) & (lane < 2 * RPW), ll_s, zero)
    for r in range(RPW):
        base = r * WPR
        cps[r].wait()
        first = vb[pl.ds(base, L)]

        def quad(c, acc, base=base):
            off = base + c * (4 * L)
            return (acc + edge_group(off) + edge_group(off + L)
                    + edge_group(off + 2 * L) + edge_group(off + 3 * L))

        acc = lax.fori_loop(0, NCHUNK // 4 - 1, quad, zero)
        off = base + WPR - 4 * L
        acc = acc + edge_group(off) + edge_group(off + L) + edge_group(off + 2 * L)
        off = off + 3 * L
        d0 = vb[pl.ds(off, L)] - vb[pl.ds(off + 2, L)]
        d1 = vb[pl.ds(off + 1, L)] - vb[pl.ds(off + 3, L)]
        d0 = jnp.where(lane == L - 2, vb[pl.ds(off, L)] - first[0], d0)
        d1 = jnp.where(lane == L - 2, vb[pl.ds(off + 1, L)] - first[1], d1)
        acc = acc + jnp.where(even, _vsqrt(d0 * d0 + d1 * d1), 0.0)
        res = jnp.where(lane == r, jnp.sum(acc), res)

    ov[...] = res
    pltpu.sync_copy(ov, out_hbm.at[wid])


@functools.partial(
    pl.kernel,
    out_type=jax.ShapeDtypeStruct((NW, L), jnp.float32),
    mesh=plsc.VectorSubcoreMesh(
        core_axis_name="c", subcore_axis_name="s", num_cores=NC, num_subcores=NS
    ),
    scratch_types=(
        pltpu.VMEM((RPW * WPR + L,), jnp.float32),
        pltpu.VMEM((L,), jnp.float32),
    ) + (pltpu.SemaphoreType.DMA,) * RPW,
    compiler_params=pltpu.CompilerParams(
        needs_layout_passes=False,
        use_tc_tiling_on_sc=False,
        skip_device_barrier=True,
        disable_bounds_checks=True,
        disable_semaphore_checks=True,
    ),
)
def _tour_kernel(in_hbm, out_hbm, vb, ov, *sems):
    _tour_body(in_hbm, out_hbm, vb, ov, *sems)


def kernel(input):
    out = _tour_kernel(input.reshape(B, WPR))
    return out[:, :RPW].reshape(B), out[:, RPW:2 * RPW].reshape(B)


# speedup vs baseline: 1.0300x; 1.0054x over previous
import functools

import jax
import jax.numpy as jnp
from jax import lax
from jax.experimental import pallas as pl
from jax.experimental.pallas import tpu as pltpu
from jax.experimental.pallas import tpu_sc as plsc

B = 128
N = 512
L = 16
NC = 2
NS = 16
NW = NC * NS
RPW = B // NW
WPR = 2 * N
NCHUNK = WPR // L

_LN2 = 0.6931471805599453


def _vlog(k_f32):
    i = plsc.bitcast(k_f32, jnp.int32)
    e = (i >> 23) - 127
    m = plsc.bitcast((i & 0x007FFFFF) | 0x3F800000, jnp.float32)
    t = (m - 1.0) / (m + 1.0)
    t2 = t * t
    p = 1.0 / 11.0
    for c in (1.0 / 9.0, 1.0 / 7.0, 1.0 / 5.0, 1.0 / 3.0, 1.0):
        p = p * t2 + c
    return e.astype(jnp.float32) * _LN2 + 2.0 * t * p


def _vsqrt(v):
    i = plsc.bitcast(v, jnp.int32)
    y = plsc.bitcast(0x5F3759DF - (i >> 1), jnp.float32)
    for _ in range(2):
        y = y * (1.5 - 0.5 * v * y * y)
    return v * y


def _tour_body(in_hbm, out_hbm, vb, ov, *sems):
    wid = lax.axis_index("c") * NS + lax.axis_index("s")
    cps = [
        pltpu.async_copy(
            in_hbm.at[wid * RPW + r], vb.at[pl.ds(r * WPR, WPR)], sems[r]
        )
        for r in range(RPW)
    ]

    lane = lax.iota(jnp.int32, L)
    zero = jnp.zeros((L,), jnp.float32)
    even = (lane & 1) == 0
    vb[pl.ds(RPW * WPR, L)] = zero

    ll_acc = zero
    lane_f = lane.astype(jnp.float32)
    for c in range(N // (8 * L)):
        s = (c * L + lane_f) * 8.0 + 1.0
        p = s
        for j in range(1, 8):
            p = p * (s + j)
        ll_acc = ll_acc + _vlog(p)
    ll_s = -jnp.sum(ll_acc)

    def edge_group(off):
        d0 = vb[pl.ds(off, L)] - vb[pl.ds(off + 2, L)]
        d1 = vb[pl.ds(off + 1, L)] - vb[pl.ds(off + 3, L)]
        return jnp.where(even, _vsqrt(d0 * d0 + d1 * d1), 0.0)

    res = jnp.where((lane >= RPW) & (lane < 2 * RPW), ll_s, zero)
    for r in range(RPW):
        base = r * WPR
        cps[r].wait()
        first = vb[pl.ds(base, L)]

        def quad(c, acc, base=base):
            off = base + c * (4 * L)
            return (acc + edge_group(off) + edge_group(off + L)
                    + edge_group(off + 2 * L) + edge_group(off + 3 * L))

        acc = lax.fori_loop(0, NCHUNK // 4 - 1, quad, zero)
        off = base + WPR - 4 * L
        acc = acc + edge_group(off) + edge_group(off + L) + edge_group(off + 2 * L)
        off = off + 3 * L
        d0 = vb[pl.ds(off, L)] - vb[pl.ds(off + 2, L)]
        d1 = vb[pl.ds(off + 1, L)] - vb[pl.ds(off + 3, L)]
        d0 = jnp.where(lane == L - 2, vb[pl.ds(off, L)] - first[0], d0)
        d1 = jnp.where(lane == L - 2, vb[pl.ds(off + 1, L)] - first[1], d1)
        acc = acc + jnp.where(even, _vsqrt(d0 * d0 + d1 * d1), 0.0)
        res = jnp.where(lane == r, jnp.sum(acc), res)

    ov[...] = res
    pltpu.sync_copy(ov, out_hbm.at[wid])


@functools.partial(
    pl.kernel,
    out_type=jax.ShapeDtypeStruct((NW, L), jnp.float32),
    mesh=plsc.VectorSubcoreMesh(
        core_axis_name="c", subcore_axis_name="s", num_cores=NC, num_subcores=NS
    ),
    scratch_types=(
        pltpu.VMEM((RPW * WPR + L,), jnp.float32),
        pltpu.VMEM((L,), jnp.float32),
    ) + (pltpu.SemaphoreType.DMA,) * RPW,
    compiler_params=pltpu.CompilerParams(
        needs_layout_passes=False,
        skip_device_barrier=True,
        disable_bounds_checks=True,
        disable_semaphore_checks=True,
    ),
)
def _tour_kernel(in_hbm, out_hbm, vb, ov, *sems):
    _tour_body(in_hbm, out_hbm, vb, ov, *sems)


def kernel(input):
    out = _tour_kernel(input.reshape(B, WPR))
    return out[:, :RPW].reshape(B), out[:, RPW:2 * RPW].reshape(B)
